# baseline (speedup 1.0000x reference)
import functools

import jax
import jax.numpy as jnp
from jax import lax
from jax.experimental import pallas as pl
from jax.experimental.pallas import tpu as pltpu
from jax.experimental.pallas import tpu_sc as plsc

N_NODES = 10000
N_EDGES = 160000
HID = 128
CHUNK = 128
N_CHUNKS = N_EDGES // CHUNK
NC, NS = 2, 16
LANES = 16
NPAD = 10240

_mesh = plsc.VectorSubcoreMesh(core_axis_name="c", subcore_axis_name="s")

EPT = N_EDGES // (NC * NS)
FULL = EPT // CHUNK
TAIL = EPT - FULL * CHUNK


def _sc_edge_body(a_h, b_h, row_h, col_h, e1_h, ibuf, cbuf, buf_a, buf_b,
                  sa0, sb0, sa1, sb1, sw0, sw1):
    c = lax.axis_index("c")
    s = lax.axis_index("s")
    wid = s * NC + c
    base = wid * EPT

    pltpu.sync_copy(row_h.at[pl.ds(base, EPT)], ibuf)
    pltpu.sync_copy(col_h.at[pl.ds(base, EPT)], cbuf)

    def start(t, slot, sa, sb):
        pltpu.async_copy(a_h.at[ibuf.at[pl.ds(t * CHUNK, CHUNK)]],
                         buf_a.at[pl.ds(slot * CHUNK, CHUNK)], sa)
        pltpu.async_copy(b_h.at[cbuf.at[pl.ds(t * CHUNK, CHUNK)]],
                         buf_b.at[pl.ds(slot * CHUNK, CHUNK)], sb)

    def wait(slot, sa, sb):
        pltpu.make_async_copy(a_h.at[pl.ds(0, CHUNK)],
                              buf_a.at[pl.ds(slot * CHUNK, CHUNK)],
                              sa).wait()
        pltpu.make_async_copy(b_h.at[pl.ds(0, CHUNK)],
                              buf_b.at[pl.ds(slot * CHUNK, CHUNK)],
                              sb).wait()

    def add_rows(slot):
        def add_row(r, inner):
            rr = slot * CHUNK + r
            for f in range(HID // LANES):
                sl = pl.ds(f * LANES, LANES)
                buf_a[rr, sl] = buf_a[rr, sl] + buf_b[rr, sl]
            return inner

        lax.fori_loop(0, CHUNK, add_row, 0)

    def startw(t, slot, sw):
        pltpu.async_copy(buf_a.at[pl.ds(slot * CHUNK, CHUNK)],
                         e1_h.at[pl.ds(base + t * CHUNK, CHUNK)], sw)

    def waitw(slot, sw):
        pltpu.make_async_copy(buf_a.at[pl.ds(slot * CHUNK, CHUNK)],
                              e1_h.at[pl.ds(0, CHUNK)], sw).wait()

    start(0, 0, sa0, sb0)
    start(1, 1, sa1, sb1)

    def pair(u, carry):
        t0 = 2 * u
        wait(0, sa0, sb0)
        add_rows(0)
        startw(t0, 0, sw0)
        wait(1, sa1, sb1)
        add_rows(1)
        startw(t0 + 1, 1, sw1)
        waitw(0, sw0)
        start(t0 + 2, 0, sa0, sb0)

        @pl.when(u < (FULL - 1) // 2 - 1)
        def _():
            waitw(1, sw1)
            start(t0 + 3, 1, sa1, sb1)

        return carry

    lax.fori_loop(0, (FULL - 1) // 2, pair, 0)
    wait(0, sa0, sb0)
    add_rows(0)
    startw(FULL - 1, 0, sw0)
    waitw(1, sw1)

    toff = FULL * CHUNK
    pltpu.async_copy(a_h.at[ibuf.at[pl.ds(toff, TAIL)]],
                     buf_a.at[pl.ds(CHUNK, TAIL)], sa1)
    pltpu.async_copy(b_h.at[cbuf.at[pl.ds(toff, TAIL)]],
                     buf_b.at[pl.ds(CHUNK, TAIL)], sb1)
    pltpu.make_async_copy(a_h.at[pl.ds(0, TAIL)],
                          buf_a.at[pl.ds(CHUNK, TAIL)], sa1).wait()
    pltpu.make_async_copy(b_h.at[pl.ds(0, TAIL)],
                          buf_b.at[pl.ds(CHUNK, TAIL)], sb1).wait()

    def add_row_t(r, inner):
        rr = CHUNK + r
        for f in range(HID // LANES):
            sl = pl.ds(f * LANES, LANES)
            buf_a[rr, sl] = buf_a[rr, sl] + buf_b[rr, sl]
        return inner

    lax.fori_loop(0, TAIL, add_row_t, 0)
    pltpu.sync_copy(buf_a.at[pl.ds(CHUNK, TAIL)],
                    e1_h.at[pl.ds(base + toff, TAIL)])
    waitw(0, sw0)


_sc_edge = pl.kernel(
    _sc_edge_body,
    out_type=jax.ShapeDtypeStruct((N_EDGES, HID), jnp.float32),
    mesh=_mesh,
    scratch_types=[
        pltpu.VMEM((EPT,), jnp.int32),
        pltpu.VMEM((EPT,), jnp.int32),
        pltpu.VMEM((2 * CHUNK, HID), jnp.float32),
        pltpu.VMEM((2 * CHUNK, HID), jnp.float32),
        pltpu.SemaphoreType.DMA,
        pltpu.SemaphoreType.DMA,
        pltpu.SemaphoreType.DMA,
        pltpu.SemaphoreType.DMA,
        pltpu.SemaphoreType.DMA,
        pltpu.SemaphoreType.DMA,
    ],
)


def _sc_deg_body(ew_h, col_h, out_h, col2, tailc, ewb, zdbuf, obuf, sem,
                 shared):
    c = lax.axis_index("c")
    s = lax.axis_index("s")
    wid = s * NC + c
    base = wid * EPT
    zero16 = jnp.zeros((LANES,), jnp.float32)
    w = NPAD // NS

    def zbody(i, carry):
        zdbuf[pl.ds(i * LANES, LANES)] = zero16
        return carry

    lax.fori_loop(0, w // LANES, zbody, 0)
    pltpu.sync_copy(zdbuf, shared.at[pl.ds(s * w, w)])

    for t in range(FULL):
        pltpu.sync_copy(col_h.at[pl.ds(base + t * CHUNK, CHUNK)], col2.at[t])
    pltpu.sync_copy(col_h.at[pl.ds(base + FULL * CHUNK, TAIL)], tailc.at[0])
    pltpu.sync_copy(ew_h.at[pl.ds(base, EPT)], ewb)
    plsc.subcore_barrier()

    def fire(t, carry):
        pltpu.async_copy(ewb.at[pl.ds(t * CHUNK, CHUNK)],
                         shared.at[col2.at[t]], sem, add=True)
        return carry

    lax.fori_loop(0, FULL, fire, 0)
    pltpu.async_copy(ewb.at[pl.ds(FULL * CHUNK, TAIL)],
                     shared.at[tailc.at[0]], sem, add=True)

    def drain(t, carry):
        pltpu.make_async_copy(col_h.at[pl.ds(0, CHUNK)], col2.at[0],
                              sem).wait()
        return carry

    lax.fori_loop(0, FULL, drain, 0)
    pltpu.make_async_copy(col_h.at[pl.ds(0, TAIL)], tailc.at[0], sem).wait()
    plsc.subcore_barrier()

    @pl.when(s < 10)
    def _():
        pltpu.sync_copy(shared.at[pl.ds(s * 1000, 1000)], obuf)
        pltpu.sync_copy(obuf, out_h.at[pl.ds(c * N_NODES + s * 1000, 1000)])


_sc_deg = pl.kernel(
    _sc_deg_body,
    out_type=jax.ShapeDtypeStruct((NC * N_NODES,), jnp.float32),
    mesh=_mesh,
    scratch_types=[
        pltpu.VMEM((FULL, CHUNK), jnp.int32),
        pltpu.VMEM((1, TAIL), jnp.int32),
        pltpu.VMEM((EPT,), jnp.float32),
        pltpu.VMEM((NPAD // NS,), jnp.float32),
        pltpu.VMEM((1000,), jnp.float32),
        pltpu.SemaphoreType.DMA,
        pltpu.VMEM_SHARED((NPAD,), jnp.float32),
    ],
)


def _sc_msg_body(y_h, row_h, col_h, ew_h, out_h, ibuf, col2, tailc, ewb,
                 ybuf, sy0, sy1, sc0, sc1, shared):
    c = lax.axis_index("c")
    s = lax.axis_index("s")
    wid = s * NC + c
    base = wid * EPT
    zero16 = jnp.zeros((LANES,), jnp.float32)

    def zbody(r, carry):
        for f in range(HID // LANES):
            ybuf[r, pl.ds(f * LANES, LANES)] = zero16
        return carry

    lax.fori_loop(0, CHUNK, zbody, 0)

    @pl.when(s < 10)
    def _():
        for k in range(7):
            pltpu.sync_copy(ybuf.at[pl.ds(0, CHUNK)],
                            shared.at[pl.ds(s * 1000 + k * CHUNK, CHUNK)])
        pltpu.sync_copy(ybuf.at[pl.ds(0, 104)],
                        shared.at[pl.ds(s * 1000 + 896, 104)])

    pltpu.sync_copy(row_h.at[pl.ds(base, EPT)], ibuf)
    for t in range(FULL):
        pltpu.sync_copy(col_h.at[pl.ds(base + t * CHUNK, CHUNK)], col2.at[t])
    pltpu.sync_copy(col_h.at[pl.ds(base + FULL * CHUNK, TAIL)], tailc.at[0])
    pltpu.sync_copy(ew_h.at[pl.ds(base, EPT)], ewb.at[pl.ds(0, EPT)])
    plsc.subcore_barrier()

    def start(t, slot, sy):
        pltpu.async_copy(y_h.at[ibuf.at[pl.ds(t * CHUNK, CHUNK)]],
                         ybuf.at[pl.ds(slot * CHUNK, CHUNK)], sy)

    def wait(slot, sy):
        pltpu.make_async_copy(y_h.at[pl.ds(0, CHUNK)],
                              ybuf.at[pl.ds(slot * CHUNK, CHUNK)],
                              sy).wait()

    def scale(t, slot):
        def scale_grp(g, inner):
            wgrp = ewb[pl.ds(t * CHUNK + g * LANES, LANES)]
            for l in range(LANES):
                w = wgrp[l]
                rr = slot * CHUNK + g * LANES + l
                for f in range(HID // LANES):
                    sl = pl.ds(f * LANES, LANES)
                    ybuf[rr, sl] = ybuf[rr, sl] * w
            return inner

        lax.fori_loop(0, CHUNK // LANES, scale_grp, 0)

    def startsc(t, slot, sc):
        pltpu.async_copy(ybuf.at[pl.ds(slot * CHUNK, CHUNK)],
                         shared.at[col2.at[t]], sc, add=True)

    def waitsc(slot, sc):
        pltpu.make_async_copy(y_h.at[pl.ds(0, CHUNK)],
                              ybuf.at[pl.ds(slot * CHUNK, CHUNK)],
                              sc).wait()

    start(0, 0, sy0)
    start(1, 1, sy1)

    def pair(u, carry):
        t0 = 2 * u
        wait(0, sy0)
        scale(t0, 0)
        startsc(t0, 0, sc0)
        wait(1, sy1)
        scale(t0 + 1, 1)
        startsc(t0 + 1, 1, sc1)
        waitsc(0, sc0)
        start(t0 + 2, 0, sy0)

        @pl.when(u < (FULL - 1) // 2 - 1)
        def _():
            waitsc(1, sc1)
            start(t0 + 3, 1, sy1)

        return carry

    lax.fori_loop(0, (FULL - 1) // 2, pair, 0)
    wait(0, sy0)
    scale(FULL - 1, 0)
    startsc(FULL - 1, 0, sc0)
    waitsc(1, sc1)

    toff = FULL * CHUNK
    pltpu.async_copy(y_h.at[ibuf.at[pl.ds(toff, TAIL)]],
                     ybuf.at[pl.ds(CHUNK, TAIL)], sy1)
    pltpu.make_async_copy(y_h.at[pl.ds(0, TAIL)],
                          ybuf.at[pl.ds(CHUNK, TAIL)], sy1).wait()
    wgrp = ewb[pl.ds(toff, LANES)]
    for l in range(TAIL):
        w = wgrp[l]
        rr = CHUNK + l
        for f in range(HID // LANES):
            sl = pl.ds(f * LANES, LANES)
            ybuf[rr, sl] = ybuf[rr, sl] * w
    pltpu.sync_copy(ybuf.at[pl.ds(CHUNK, TAIL)], shared.at[tailc.at[0]],
                    add=True)
    waitsc(0, sc0)
    plsc.subcore_barrier()

    @pl.when(s < 10)
    def _():
        for k in range(7):
            off = s * 1000 + k * CHUNK
            pltpu.sync_copy(shared.at[pl.ds(off, CHUNK)],
                            ybuf.at[pl.ds(0, CHUNK)])
            pltpu.sync_copy(ybuf.at[pl.ds(0, CHUNK)],
                            out_h.at[c, pl.ds(off, CHUNK)])
        off = s * 1000 + 896
        pltpu.sync_copy(shared.at[pl.ds(off, 104)], ybuf.at[pl.ds(0, 104)])
        pltpu.sync_copy(ybuf.at[pl.ds(0, 104)],
                        out_h.at[c, pl.ds(off, 104)])


_sc_msg = pl.kernel(
    _sc_msg_body,
    out_type=jax.ShapeDtypeStruct((NC, N_NODES, HID), jnp.float32),
    mesh=_mesh,
    scratch_types=[
        pltpu.VMEM((EPT,), jnp.int32),
        pltpu.VMEM((FULL, CHUNK), jnp.int32),
        pltpu.VMEM((1, TAIL), jnp.int32),
        pltpu.VMEM((EPT + LANES,), jnp.float32),
        pltpu.VMEM((2 * CHUNK, HID), jnp.float32),
        pltpu.SemaphoreType.DMA,
        pltpu.SemaphoreType.DMA,
        pltpu.SemaphoreType.DMA,
        pltpu.SemaphoreType.DMA,
        pltpu.VMEM_SHARED((N_NODES, HID), jnp.float32),
    ],
)


def _dot(a, b):
    return jnp.dot(a, b, preferred_element_type=jnp.float32,
                   precision=lax.Precision.HIGHEST)


_RB = 2000


def _pack_bf16(v):
    vb = v.astype(jnp.bfloat16)
    lo = lax.bitcast_convert_type(vb[:, :64], jnp.uint16).astype(jnp.uint32)
    hi = lax.bitcast_convert_type(vb[:, 64:], jnp.uint16).astype(jnp.uint32)
    return lax.bitcast_convert_type(lo | (hi << 16), jnp.int32)


def _unpack_bf16(w):
    wu = lax.bitcast_convert_type(w, jnp.uint32)
    lo = lax.bitcast_convert_type((wu & 0xFFFF).astype(jnp.uint16),
                                  jnp.bfloat16)
    hi = lax.bitcast_convert_type((wu >> 16).astype(jnp.uint16),
                                  jnp.bfloat16)
    return jnp.concatenate([lo, hi], axis=1).astype(jnp.float32)


def _tc1_body(x_ref, w1_ref, b1_ref, w2_ref, b2_ref, wa_ref, ba_ref, wb_ref,
              gw1_ref, h_ref, a_ref, b_ref, xw1_ref):
    x = x_ref[...]
    hm = jnp.maximum(
        _dot(x, w1_ref[...])
        + b1_ref[...], 0.0)
    h = jnp.maximum(
        _dot(hm, w2_ref[...])
        + b2_ref[...], 0.0)
    h_ref[...] = h
    a_ref[...] = (_dot(h, wa_ref[...])
                  + ba_ref[...])
    b_ref[...] = _dot(h, wb_ref[...])
    xw1_ref[...] = _dot(h, gw1_ref[...])


def _tc1(x, w1f, b1f, w2, b2, wa, ba, wb, gw1):
    nb = N_NODES // _RB
    full = lambda shape: pl.BlockSpec(shape, lambda i: (0, 0))
    row_blk = lambda w: pl.BlockSpec((_RB, w), lambda i: (i, 0))
    return pl.pallas_call(
        _tc1_body,
        grid=(nb,),
        in_specs=[
            row_blk(HID), full((HID, 64)), full((1, 64)), full((64, HID)),
            full((1, HID)), full((HID, HID)), full((1, HID)),
            full((HID, HID)), full((HID, HID)),
        ],
        out_specs=[row_blk(HID), row_blk(HID), row_blk(HID), row_blk(HID)],
        out_shape=[jax.ShapeDtypeStruct((N_NODES, HID), jnp.float32)] * 4,
    )(x, w1f, b1f, w2, b2, wa, ba, wb, gw1)


_BE = 1280


def _tc2_body(e1_ref, ew2_ref, eb2_ref, ew3_ref, eb3_ref, out_ref):
    e1 = jnp.maximum(e1_ref[...], 0.0)
    e2 = jnp.maximum(
        jnp.dot(e1, ew2_ref[...], preferred_element_type=jnp.float32)
        + eb2_ref[...], 0.0)
    v = jnp.sum(e2 * ew3_ref[...], axis=1, keepdims=True) + eb3_ref[0, 0]
    out_ref[...] = jax.nn.sigmoid(v)


def _tc2(e1, ew2, eb2, ew3t, eb3):
    nb = N_EDGES // _BE
    full = lambda shape: pl.BlockSpec(shape, lambda i: (0, 0))
    return pl.pallas_call(
        _tc2_body,
        grid=(nb,),
        in_specs=[
            pl.BlockSpec((_BE, HID), lambda i: (i, 0)),
            full((HID, 64)), full((1, 64)), full((1, 64)), full((1, 1)),
        ],
        out_specs=pl.BlockSpec((_BE, 1), lambda i: (i, 0)),
        out_shape=jax.ShapeDtypeStruct((N_EDGES, 1), jnp.float32),
    )(e1, ew2, eb2, ew3t, eb3).reshape(N_EDGES)


def _tc3_body(p0_ref, p1_ref, xw1_ref, h_ref, gb1_ref, dinv_ref, y1_ref,
              aux1_ref):
    deg = 1.0 + p0_ref[...] + p1_ref[...]
    dinv = jnp.where(deg > 0, lax.rsqrt(jnp.maximum(deg, 1e-12)), 0.0)
    dinv_ref[...] = dinv
    xw = xw1_ref[...]
    y1_ref[...] = dinv * xw
    aux1_ref[...] = h_ref[...] + gb1_ref[...] + dinv * dinv * xw


def _tc3(p0, p1, xw1, h, gb1):
    nb = N_NODES // _RB
    col_blk = pl.BlockSpec((_RB, 1), lambda i: (i, 0))
    row_blk = pl.BlockSpec((_RB, HID), lambda i: (i, 0))
    full = lambda shape: pl.BlockSpec(shape, lambda i: (0, 0))
    return pl.pallas_call(
        _tc3_body,
        grid=(nb,),
        in_specs=[col_blk, col_blk, row_blk, row_blk, full((1, HID))],
        out_specs=[col_blk, row_blk, row_blk],
        out_shape=[
            jax.ShapeDtypeStruct((N_NODES, 1), jnp.float32),
            jax.ShapeDtypeStruct((N_NODES, HID), jnp.float32),
            jax.ShapeDtypeStruct((N_NODES, HID), jnp.float32),
        ],
    )(p0, p1, xw1, h, gb1)


def _tc4_body(aux1_ref, q0_ref, q1_ref, dinv_ref, gw2_ref, gb2_ref, y2_ref,
              aux2_ref):
    dinv = dinv_ref[...]
    h1 = jnp.maximum(aux1_ref[...] + dinv * (q0_ref[...] + q1_ref[...]), 0.0)
    xw2 = _dot(h1, gw2_ref[...])
    y2_ref[...] = dinv * xw2
    aux2_ref[...] = h1 + gb2_ref[...] + dinv * dinv * xw2


def _tc4(aux1, q0, q1, dinv, gw2, gb2):
    nb = N_NODES // _RB
    col_blk = pl.BlockSpec((_RB, 1), lambda i: (i, 0))
    row_blk = pl.BlockSpec((_RB, HID), lambda i: (i, 0))
    full = lambda shape: pl.BlockSpec(shape, lambda i: (0, 0))
    return pl.pallas_call(
        _tc4_body,
        grid=(nb,),
        in_specs=[row_blk, row_blk, row_blk, col_blk, full((HID, HID)),
                  full((1, HID))],
        out_specs=[row_blk, row_blk],
        out_shape=[jax.ShapeDtypeStruct((N_NODES, HID), jnp.float32)] * 2,
    )(aux1, q0, q1, dinv, gw2, gb2)


def _tc5_body(aux2_ref, r0_ref, r1_ref, dinv_ref, pw1_ref, pb1_ref, pw2_ref,
              pb2_ref, pw3_ref, pb3_ref, out_ref, sacc, macc):
    i = pl.program_id(0)
    dinv = dinv_ref[...]
    h2 = jnp.maximum(aux2_ref[...] + dinv * (r0_ref[...] + r1_ref[...]), 0.0)

    @pl.when(i == 0)
    def _():
        sacc[...] = jnp.zeros_like(sacc)
        macc[...] = jnp.full_like(macc, -jnp.inf)

    sacc[...] = sacc[...] + jnp.sum(h2, axis=0, keepdims=True)
    macc[...] = jnp.maximum(macc[...], jnp.max(h2, axis=0, keepdims=True))

    @pl.when(i == pl.num_programs(0) - 1)
    def _():
        g = (sacc[...] / N_NODES + macc[...]) * 0.5
        o1 = jnp.maximum(
            _dot(g, pw1_ref[...])
            + pb1_ref[...], 0.0)
        o2 = jnp.maximum(
            _dot(o1, pw2_ref[...])
            + pb2_ref[...], 0.0)
        out_ref[...] = (_dot(o2, pw3_ref[...])
                        + pb3_ref[...])


def _tc5(aux2, r0, r1, dinv, pw1, pb1, pw2, pb2, pw3, pb3):
    nb = N_NODES // _RB
    col_blk = pl.BlockSpec((_RB, 1), lambda i: (i, 0))
    row_blk = pl.BlockSpec((_RB, HID), lambda i: (i, 0))
    full = lambda shape: pl.BlockSpec(shape, lambda i: (0, 0))
    return pl.pallas_call(
        _tc5_body,
        grid=(nb,),
        in_specs=[row_blk, row_blk, row_blk, col_blk,
                  full((HID, 64)), full((1, 64)), full((64, 32)),
                  full((1, 32)), full((32, 1)), full((1, 1))],
        out_specs=pl.BlockSpec((1, 1), lambda i: (0, 0)),
        out_shape=jax.ShapeDtypeStruct((1, 1), jnp.float32),
        scratch_shapes=[
            pltpu.VMEM((1, HID), jnp.float32),
            pltpu.VMEM((1, HID), jnp.float32),
        ],
    )(aux2, r0, r1, dinv, pw1, pb1, pw2, pb2, pw3, pb3)


def kernel(x, edge_index, batch, W1, b1, g1, be1, W2, b2, Ew1, Eb1, Eg, Ebe,
           Ew2, Eb2, Ew3, Eb3, GW1, Gb1, GW2, Gb2, Pw1, Pb1, Pw2, Pb2, Pw3,
           Pb3):
    s = 1.0 / jnp.sqrt(jnp.float32(1.0 + 1e-5))
    w1f = W1 * (g1 * s)[None, :]
    b1f = (b1 * g1 * s + be1)[None, :]
    ewa = Ew1[:HID] * (Eg * s)[None, :]
    ewb = Ew1[HID:] * (Eg * s)[None, :]
    ebf = (Eb1 * Eg * s + Ebe)[None, :]
    row = edge_index[0]
    col = edge_index[1]

    h, a_t, b_t, xw1 = _tc1(x, w1f, b1f, W2, b2[None], ewa, ebf, ewb, GW1)
    e1 = _sc_edge(a_t, b_t, row, col)
    ew = _tc2(e1, Ew2, Eb2[None], Ew3.T, Eb3[None])
    degp = _sc_deg(ew, col)
    p0 = degp[:N_NODES][:, None]
    p1 = degp[N_NODES:][:, None]
    dinv, y1, aux1 = _tc3(p0, p1, xw1, h, Gb1[None])
    q = _sc_msg(y1, row, col, ew)
    y2, aux2 = _tc4(aux1, q[0], q[1], dinv, GW2, Gb2[None])
    r = _sc_msg(y2, row, col, ew)
    return _tc5(aux2, r[0], r[1], dinv, Pw1, Pb1[None], Pw2, Pb2[None], Pw3,
                Pb3[None])


# speedup vs baseline: 1.0557x; 1.0557x over previous
import functools

import jax
import jax.numpy as jnp
from jax import lax
from jax.experimental import pallas as pl
from jax.experimental.pallas import tpu as pltpu
from jax.experimental.pallas import tpu_sc as plsc

N_NODES = 10000
N_EDGES = 160000
HID = 128
CHUNK = 128
N_CHUNKS = N_EDGES // CHUNK
NC, NS = 2, 16
LANES = 16
NPAD = 10240

_mesh = plsc.VectorSubcoreMesh(core_axis_name="c", subcore_axis_name="s")

EPT = N_EDGES // (NC * NS)
FULL = EPT // CHUNK
TAIL = EPT - FULL * CHUNK


def _sc_edge_body(a_h, b_h, row_h, col_h, e1_h, ibuf, cbuf, buf_a, buf_b,
                  sa0, sb0, sa1, sb1):
    c = lax.axis_index("c")
    s = lax.axis_index("s")
    wid = s * NC + c
    base = wid * EPT

    pltpu.sync_copy(row_h.at[pl.ds(base, EPT)], ibuf)
    pltpu.sync_copy(col_h.at[pl.ds(base, EPT)], cbuf)

    def start(t, slot, sa, sb):
        pltpu.async_copy(a_h.at[ibuf.at[pl.ds(t * CHUNK, CHUNK)]],
                         buf_a.at[pl.ds(slot * CHUNK, CHUNK)], sa)
        pltpu.async_copy(b_h.at[cbuf.at[pl.ds(t * CHUNK, CHUNK)]],
                         buf_b.at[pl.ds(slot * CHUNK, CHUNK)], sb)

    def wait(slot, sa, sb):
        pltpu.make_async_copy(a_h.at[pl.ds(0, CHUNK)],
                              buf_a.at[pl.ds(slot * CHUNK, CHUNK)],
                              sa).wait()
        pltpu.make_async_copy(b_h.at[pl.ds(0, CHUNK)],
                              buf_b.at[pl.ds(slot * CHUNK, CHUNK)],
                              sb).wait()

    def process(t, slot):
        def add_row(r, inner):
            rr = slot * CHUNK + r
            for f in range(HID // LANES):
                sl = pl.ds(f * LANES, LANES)
                buf_a[rr, sl] = buf_a[rr, sl] + buf_b[rr, sl]
            return inner

        lax.fori_loop(0, CHUNK, add_row, 0)
        pltpu.sync_copy(buf_a.at[pl.ds(slot * CHUNK, CHUNK)],
                        e1_h.at[pl.ds(base + t * CHUNK, CHUNK)])

    start(0, 0, sa0, sb0)

    def pair(u, carry):
        t0 = 2 * u
        start(t0 + 1, 1, sa1, sb1)
        wait(0, sa0, sb0)
        process(t0, 0)
        start(t0 + 2, 0, sa0, sb0)
        wait(1, sa1, sb1)
        process(t0 + 1, 1)
        return carry

    lax.fori_loop(0, (FULL - 1) // 2, pair, 0)
    wait(0, sa0, sb0)
    process(FULL - 1, 0)

    toff = FULL * CHUNK
    pltpu.async_copy(a_h.at[ibuf.at[pl.ds(toff, TAIL)]],
                     buf_a.at[pl.ds(0, TAIL)], sa0)
    pltpu.async_copy(b_h.at[cbuf.at[pl.ds(toff, TAIL)]],
                     buf_b.at[pl.ds(0, TAIL)], sb0)
    pltpu.make_async_copy(a_h.at[pl.ds(0, TAIL)], buf_a.at[pl.ds(0, TAIL)],
                          sa0).wait()
    pltpu.make_async_copy(b_h.at[pl.ds(0, TAIL)], buf_b.at[pl.ds(0, TAIL)],
                          sb0).wait()

    def add_row_t(r, inner):
        for f in range(HID // LANES):
            sl = pl.ds(f * LANES, LANES)
            buf_a[r, sl] = buf_a[r, sl] + buf_b[r, sl]
        return inner

    lax.fori_loop(0, TAIL, add_row_t, 0)
    pltpu.sync_copy(buf_a.at[pl.ds(0, TAIL)],
                    e1_h.at[pl.ds(base + toff, TAIL)])


_sc_edge = pl.kernel(
    _sc_edge_body,
    out_type=jax.ShapeDtypeStruct((N_EDGES, HID), jnp.float32),
    mesh=_mesh,
    scratch_types=[
        pltpu.VMEM((EPT,), jnp.int32),
        pltpu.VMEM((EPT,), jnp.int32),
        pltpu.VMEM((2 * CHUNK, HID), jnp.float32),
        pltpu.VMEM((2 * CHUNK, HID), jnp.float32),
        pltpu.SemaphoreType.DMA,
        pltpu.SemaphoreType.DMA,
        pltpu.SemaphoreType.DMA,
        pltpu.SemaphoreType.DMA,
    ],
)


def _sc_deg_body(ew_h, col_h, out_h, col2, tailc, ewb, zdbuf, obuf, sem,
                 shared):
    c = lax.axis_index("c")
    s = lax.axis_index("s")
    wid = s * NC + c
    base = wid * EPT
    zero16 = jnp.zeros((LANES,), jnp.float32)
    w = NPAD // NS

    def zbody(i, carry):
        zdbuf[pl.ds(i * LANES, LANES)] = zero16
        return carry

    lax.fori_loop(0, w // LANES, zbody, 0)
    pltpu.sync_copy(zdbuf, shared.at[pl.ds(s * w, w)])

    for t in range(FULL):
        pltpu.sync_copy(col_h.at[pl.ds(base + t * CHUNK, CHUNK)], col2.at[t])
    pltpu.sync_copy(col_h.at[pl.ds(base + FULL * CHUNK, TAIL)], tailc.at[0])
    pltpu.sync_copy(ew_h.at[pl.ds(base, EPT)], ewb)
    plsc.subcore_barrier()

    def fire(t, carry):
        pltpu.async_copy(ewb.at[pl.ds(t * CHUNK, CHUNK)],
                         shared.at[col2.at[t]], sem, add=True)
        return carry

    lax.fori_loop(0, FULL, fire, 0)
    pltpu.async_copy(ewb.at[pl.ds(FULL * CHUNK, TAIL)],
                     shared.at[tailc.at[0]], sem, add=True)

    def drain(t, carry):
        pltpu.make_async_copy(col_h.at[pl.ds(0, CHUNK)], col2.at[0],
                              sem).wait()
        return carry

    lax.fori_loop(0, FULL, drain, 0)
    pltpu.make_async_copy(col_h.at[pl.ds(0, TAIL)], tailc.at[0], sem).wait()
    plsc.subcore_barrier()

    @pl.when(s < 10)
    def _():
        pltpu.sync_copy(shared.at[pl.ds(s * 1000, 1000)], obuf)
        pltpu.sync_copy(obuf, out_h.at[pl.ds(c * N_NODES + s * 1000, 1000)])


_sc_deg = pl.kernel(
    _sc_deg_body,
    out_type=jax.ShapeDtypeStruct((NC * N_NODES,), jnp.float32),
    mesh=_mesh,
    scratch_types=[
        pltpu.VMEM((FULL, CHUNK), jnp.int32),
        pltpu.VMEM((1, TAIL), jnp.int32),
        pltpu.VMEM((EPT,), jnp.float32),
        pltpu.VMEM((NPAD // NS,), jnp.float32),
        pltpu.VMEM((1000,), jnp.float32),
        pltpu.SemaphoreType.DMA,
        pltpu.VMEM_SHARED((NPAD,), jnp.float32),
    ],
)


def _sc_msg_body(y_h, row_h, col_h, ew_h, out_h, ibuf, col2, tailc, ewb,
                 ybuf, sy0, sy1, shared):
    c = lax.axis_index("c")
    s = lax.axis_index("s")
    wid = s * NC + c
    base = wid * EPT
    zero16 = jnp.zeros((LANES,), jnp.float32)

    def zbody(r, carry):
        for f in range(HID // LANES):
            ybuf[r, pl.ds(f * LANES, LANES)] = zero16
        return carry

    lax.fori_loop(0, CHUNK, zbody, 0)

    @pl.when(s < 10)
    def _():
        for k in range(7):
            pltpu.sync_copy(ybuf.at[pl.ds(0, CHUNK)],
                            shared.at[pl.ds(s * 1000 + k * CHUNK, CHUNK)])
        pltpu.sync_copy(ybuf.at[pl.ds(0, 104)],
                        shared.at[pl.ds(s * 1000 + 896, 104)])

    pltpu.sync_copy(row_h.at[pl.ds(base, EPT)], ibuf)
    for t in range(FULL):
        pltpu.sync_copy(col_h.at[pl.ds(base + t * CHUNK, CHUNK)], col2.at[t])
    pltpu.sync_copy(col_h.at[pl.ds(base + FULL * CHUNK, TAIL)], tailc.at[0])
    pltpu.sync_copy(ew_h.at[pl.ds(base, EPT)], ewb.at[pl.ds(0, EPT)])
    plsc.subcore_barrier()

    def start(t, slot, sy):
        pltpu.async_copy(y_h.at[ibuf.at[pl.ds(t * CHUNK, CHUNK)]],
                         ybuf.at[pl.ds(slot * CHUNK, CHUNK)], sy)

    def wait(slot, sy):
        pltpu.make_async_copy(y_h.at[pl.ds(0, CHUNK)],
                              ybuf.at[pl.ds(slot * CHUNK, CHUNK)],
                              sy).wait()

    def process(t, slot):
        def scale_grp(g, inner):
            wgrp = ewb[pl.ds(t * CHUNK + g * LANES, LANES)]
            for l in range(LANES):
                w = wgrp[l]
                rr = slot * CHUNK + g * LANES + l
                for f in range(HID // LANES):
                    sl = pl.ds(f * LANES, LANES)
                    ybuf[rr, sl] = ybuf[rr, sl] * w
            return inner

        lax.fori_loop(0, CHUNK // LANES, scale_grp, 0)
        pltpu.sync_copy(ybuf.at[pl.ds(slot * CHUNK, CHUNK)],
                        shared.at[col2.at[t]], add=True)

    start(0, 0, sy0)

    def pair(u, carry):
        t0 = 2 * u
        start(t0 + 1, 1, sy1)
        wait(0, sy0)
        process(t0, 0)
        start(t0 + 2, 0, sy0)
        wait(1, sy1)
        process(t0 + 1, 1)
        return carry

    lax.fori_loop(0, (FULL - 1) // 2, pair, 0)
    wait(0, sy0)
    process(FULL - 1, 0)

    toff = FULL * CHUNK
    pltpu.async_copy(y_h.at[ibuf.at[pl.ds(toff, TAIL)]],
                     ybuf.at[pl.ds(0, TAIL)], sy0)
    pltpu.make_async_copy(y_h.at[pl.ds(0, TAIL)], ybuf.at[pl.ds(0, TAIL)],
                          sy0).wait()
    wgrp = ewb[pl.ds(toff, LANES)]
    for l in range(TAIL):
        w = wgrp[l]
        for f in range(HID // LANES):
            sl = pl.ds(f * LANES, LANES)
            ybuf[l, sl] = ybuf[l, sl] * w
    pltpu.sync_copy(ybuf.at[pl.ds(0, TAIL)], shared.at[tailc.at[0]],
                    add=True)
    plsc.subcore_barrier()

    @pl.when(s < 10)
    def _():
        for k in range(7):
            off = s * 1000 + k * CHUNK
            pltpu.sync_copy(shared.at[pl.ds(off, CHUNK)],
                            ybuf.at[pl.ds(0, CHUNK)])
            pltpu.sync_copy(ybuf.at[pl.ds(0, CHUNK)],
                            out_h.at[c, pl.ds(off, CHUNK)])
        off = s * 1000 + 896
        pltpu.sync_copy(shared.at[pl.ds(off, 104)], ybuf.at[pl.ds(0, 104)])
        pltpu.sync_copy(ybuf.at[pl.ds(0, 104)],
                        out_h.at[c, pl.ds(off, 104)])


_sc_msg = pl.kernel(
    _sc_msg_body,
    out_type=jax.ShapeDtypeStruct((NC, N_NODES, HID), jnp.float32),
    mesh=_mesh,
    scratch_types=[
        pltpu.VMEM((EPT,), jnp.int32),
        pltpu.VMEM((FULL, CHUNK), jnp.int32),
        pltpu.VMEM((1, TAIL), jnp.int32),
        pltpu.VMEM((EPT + LANES,), jnp.float32),
        pltpu.VMEM((2 * CHUNK, HID), jnp.float32),
        pltpu.SemaphoreType.DMA,
        pltpu.SemaphoreType.DMA,
        pltpu.VMEM_SHARED((N_NODES, HID), jnp.float32),
    ],
)


_RB = 2000


def _dot(a, b):
    return jnp.dot(a, b, preferred_element_type=jnp.float32,
                   precision=lax.Precision.HIGHEST)


def _tc1_body(x_ref, w1_ref, b1_ref, w2_ref, b2_ref, wa_ref, ba_ref, wb_ref,
              gw1_ref, h_ref, a_ref, b_ref, xw1_ref):
    x = x_ref[...]
    hm = jnp.maximum(_dot(x, w1_ref[...]) + b1_ref[...], 0.0)
    h = jnp.maximum(_dot(hm, w2_ref[...]) + b2_ref[...], 0.0)
    h_ref[...] = h
    a_ref[...] = _dot(h, wa_ref[...]) + ba_ref[...]
    b_ref[...] = _dot(h, wb_ref[...])
    xw1_ref[...] = _dot(h, gw1_ref[...])


def _tc1(x, w1f, b1f, w2, b2, wa, ba, wb, gw1):
    nb = N_NODES // _RB
    full = lambda shape: pl.BlockSpec(shape, lambda i: (0, 0))
    row_blk = lambda w: pl.BlockSpec((_RB, w), lambda i: (i, 0))
    return pl.pallas_call(
        _tc1_body,
        grid=(nb,),
        in_specs=[
            row_blk(HID), full((HID, 64)), full((1, 64)), full((64, HID)),
            full((1, HID)), full((HID, HID)), full((1, HID)),
            full((HID, HID)), full((HID, HID)),
        ],
        out_specs=[row_blk(HID), row_blk(HID), row_blk(HID), row_blk(HID)],
        out_shape=[jax.ShapeDtypeStruct((N_NODES, HID), jnp.float32)] * 4,
    )(x, w1f, b1f, w2, b2, wa, ba, wb, gw1)


_BE = 1280


def _tc2_body(e1_ref, ew2_ref, eb2_ref, ew3_ref, eb3_ref, out_ref):
    e1 = jnp.maximum(e1_ref[...], 0.0)
    e2 = jnp.maximum(
        jnp.dot(e1, ew2_ref[...], preferred_element_type=jnp.float32)
        + eb2_ref[...], 0.0)
    v = jnp.sum(e2 * ew3_ref[...], axis=1, keepdims=True) + eb3_ref[0, 0]
    out_ref[...] = jax.nn.sigmoid(v)


def _tc2(e1, ew2, eb2, ew3t, eb3):
    nb = N_EDGES // _BE
    full = lambda shape: pl.BlockSpec(shape, lambda i: (0, 0))
    return pl.pallas_call(
        _tc2_body,
        grid=(nb,),
        in_specs=[
            pl.BlockSpec((_BE, HID), lambda i: (i, 0)),
            full((HID, 64)), full((1, 64)), full((1, 64)), full((1, 1)),
        ],
        out_specs=pl.BlockSpec((_BE, 1), lambda i: (i, 0)),
        out_shape=jax.ShapeDtypeStruct((N_EDGES, 1), jnp.float32),
    )(e1, ew2, eb2, ew3t, eb3).reshape(N_EDGES)


def _tc3_body(p0_ref, p1_ref, xw1_ref, h_ref, gb1_ref, dinv_ref, y1_ref,
              aux1_ref):
    deg = 1.0 + p0_ref[...] + p1_ref[...]
    dinv = jnp.where(deg > 0, lax.rsqrt(jnp.maximum(deg, 1e-12)), 0.0)
    dinv_ref[...] = dinv
    xw = xw1_ref[...]
    y1_ref[...] = dinv * xw
    aux1_ref[...] = h_ref[...] + gb1_ref[...] + dinv * dinv * xw


def _tc3(p0, p1, xw1, h, gb1):
    nb = N_NODES // _RB
    col_blk = pl.BlockSpec((_RB, 1), lambda i: (i, 0))
    row_blk = pl.BlockSpec((_RB, HID), lambda i: (i, 0))
    full = lambda shape: pl.BlockSpec(shape, lambda i: (0, 0))
    return pl.pallas_call(
        _tc3_body,
        grid=(nb,),
        in_specs=[col_blk, col_blk, row_blk, row_blk, full((1, HID))],
        out_specs=[col_blk, row_blk, row_blk],
        out_shape=[
            jax.ShapeDtypeStruct((N_NODES, 1), jnp.float32),
            jax.ShapeDtypeStruct((N_NODES, HID), jnp.float32),
            jax.ShapeDtypeStruct((N_NODES, HID), jnp.float32),
        ],
    )(p0, p1, xw1, h, gb1)


def _tc4_body(aux1_ref, q0_ref, q1_ref, dinv_ref, gw2_ref, gb2_ref, y2_ref,
              aux2_ref):
    dinv = dinv_ref[...]
    h1 = jnp.maximum(aux1_ref[...] + dinv * (q0_ref[...] + q1_ref[...]), 0.0)
    xw2 = _dot(h1, gw2_ref[...])
    y2_ref[...] = dinv * xw2
    aux2_ref[...] = h1 + gb2_ref[...] + dinv * dinv * xw2


def _tc4(aux1, q0, q1, dinv, gw2, gb2):
    nb = N_NODES // _RB
    col_blk = pl.BlockSpec((_RB, 1), lambda i: (i, 0))
    row_blk = pl.BlockSpec((_RB, HID), lambda i: (i, 0))
    full = lambda shape: pl.BlockSpec(shape, lambda i: (0, 0))
    return pl.pallas_call(
        _tc4_body,
        grid=(nb,),
        in_specs=[row_blk, row_blk, row_blk, col_blk, full((HID, HID)),
                  full((1, HID))],
        out_specs=[row_blk, row_blk],
        out_shape=[jax.ShapeDtypeStruct((N_NODES, HID), jnp.float32)] * 2,
    )(aux1, q0, q1, dinv, gw2, gb2)


def _tc5_body(aux2_ref, r0_ref, r1_ref, dinv_ref, pw1_ref, pb1_ref, pw2_ref,
              pb2_ref, pw3_ref, pb3_ref, out_ref, sacc, macc):
    i = pl.program_id(0)
    dinv = dinv_ref[...]
    h2 = jnp.maximum(aux2_ref[...] + dinv * (r0_ref[...] + r1_ref[...]), 0.0)

    @pl.when(i == 0)
    def _():
        sacc[...] = jnp.zeros_like(sacc)
        macc[...] = jnp.full_like(macc, -jnp.inf)

    sacc[...] = sacc[...] + jnp.sum(h2, axis=0, keepdims=True)
    macc[...] = jnp.maximum(macc[...], jnp.max(h2, axis=0, keepdims=True))

    @pl.when(i == pl.num_programs(0) - 1)
    def _():
        g = (sacc[...] / N_NODES + macc[...]) * 0.5
        o1 = jnp.maximum(_dot(g, pw1_ref[...]) + pb1_ref[...], 0.0)
        o2 = jnp.maximum(_dot(o1, pw2_ref[...]) + pb2_ref[...], 0.0)
        out_ref[...] = _dot(o2, pw3_ref[...]) + pb3_ref[...]


def _tc5(aux2, r0, r1, dinv, pw1, pb1, pw2, pb2, pw3, pb3):
    nb = N_NODES // _RB
    col_blk = pl.BlockSpec((_RB, 1), lambda i: (i, 0))
    row_blk = pl.BlockSpec((_RB, HID), lambda i: (i, 0))
    full = lambda shape: pl.BlockSpec(shape, lambda i: (0, 0))
    return pl.pallas_call(
        _tc5_body,
        grid=(nb,),
        in_specs=[row_blk, row_blk, row_blk, col_blk,
                  full((HID, 64)), full((1, 64)), full((64, 32)),
                  full((1, 32)), full((32, 1)), full((1, 1))],
        out_specs=pl.BlockSpec((1, 1), lambda i: (0, 0)),
        out_shape=jax.ShapeDtypeStruct((1, 1), jnp.float32),
        scratch_shapes=[
            pltpu.VMEM((1, HID), jnp.float32),
            pltpu.VMEM((1, HID), jnp.float32),
        ],
    )(aux2, r0, r1, dinv, pw1, pb1, pw2, pb2, pw3, pb3)


def kernel(x, edge_index, batch, W1, b1, g1, be1, W2, b2, Ew1, Eb1, Eg, Ebe,
           Ew2, Eb2, Ew3, Eb3, GW1, Gb1, GW2, Gb2, Pw1, Pb1, Pw2, Pb2, Pw3,
           Pb3):
    s = 1.0 / jnp.sqrt(jnp.float32(1.0 + 1e-5))
    w1f = W1 * (g1 * s)[None, :]
    b1f = (b1 * g1 * s + be1)[None, :]
    ewa = Ew1[:HID] * (Eg * s)[None, :]
    ewb = Ew1[HID:] * (Eg * s)[None, :]
    ebf = (Eb1 * Eg * s + Ebe)[None, :]
    row = edge_index[0]
    col = edge_index[1]

    h, a_t, b_t, xw1 = _tc1(x, w1f, b1f, W2, b2[None], ewa, ebf, ewb, GW1)
    e1 = _sc_edge(a_t, b_t, row, col)
    ew = _tc2(e1, Ew2, Eb2[None], Ew3.T, Eb3[None])
    degp = _sc_deg(ew, col)
    p0 = degp[:N_NODES][:, None]
    p1 = degp[N_NODES:][:, None]
    dinv, y1, aux1 = _tc3(p0, p1, xw1, h, Gb1[None])
    q = _sc_msg(y1, row, col, ew)
    y2, aux2 = _tc4(aux1, q[0], q[1], dinv, GW2, Gb2[None])
    r = _sc_msg(y2, row, col, ew)
    return _tc5(aux2, r[0], r[1], dinv, Pw1, Pb1[None], Pw2, Pb2[None], Pw3,
                Pb3[None])


# speedup vs baseline: 1.1677x; 1.1061x over previous
import functools

import jax
import jax.numpy as jnp
from jax import lax
from jax.experimental import pallas as pl
from jax.experimental.pallas import tpu as pltpu
from jax.experimental.pallas import tpu_sc as plsc

N_NODES = 10000
N_EDGES = 160000
HID = 128
CHUNK = 128
N_CHUNKS = N_EDGES // CHUNK
NC, NS = 2, 16
LANES = 16
NPAD = 10240

_mesh = plsc.VectorSubcoreMesh(core_axis_name="c", subcore_axis_name="s")

EPT = N_EDGES // (NC * NS)
FULL = EPT // CHUNK
TAIL = EPT - FULL * CHUNK


def _sc_edge_body(a_h, b_h, row_h, col_h, e1_h, ibuf, cbuf, buf_a, buf_b,
                  sa0, sb0, sa1, sb1):
    c = lax.axis_index("c")
    s = lax.axis_index("s")
    wid = s * NC + c
    base = wid * EPT

    pltpu.sync_copy(row_h.at[pl.ds(base, EPT)], ibuf)
    pltpu.sync_copy(col_h.at[pl.ds(base, EPT)], cbuf)

    def start(t, slot, sa, sb):
        pltpu.async_copy(a_h.at[ibuf.at[pl.ds(t * CHUNK, CHUNK)]],
                         buf_a.at[pl.ds(slot * CHUNK, CHUNK)], sa)
        pltpu.async_copy(b_h.at[cbuf.at[pl.ds(t * CHUNK, CHUNK)]],
                         buf_b.at[pl.ds(slot * CHUNK, CHUNK)], sb)

    def wait(slot, sa, sb):
        pltpu.make_async_copy(a_h.at[pl.ds(0, CHUNK)],
                              buf_a.at[pl.ds(slot * CHUNK, CHUNK)],
                              sa).wait()
        pltpu.make_async_copy(b_h.at[pl.ds(0, CHUNK)],
                              buf_b.at[pl.ds(slot * CHUNK, CHUNK)],
                              sb).wait()

    def process(t, slot):
        def add_row(r, inner):
            rr = slot * CHUNK + r
            for f in range(HID // LANES):
                sl = pl.ds(f * LANES, LANES)
                buf_a[rr, sl] = buf_a[rr, sl] + buf_b[rr, sl]
            return inner

        lax.fori_loop(0, CHUNK, add_row, 0)
        pltpu.sync_copy(buf_a.at[pl.ds(slot * CHUNK, CHUNK)],
                        e1_h.at[pl.ds(base + t * CHUNK, CHUNK)])

    start(0, 0, sa0, sb0)

    def pair(u, carry):
        t0 = 2 * u
        start(t0 + 1, 1, sa1, sb1)
        wait(0, sa0, sb0)
        process(t0, 0)
        start(t0 + 2, 0, sa0, sb0)
        wait(1, sa1, sb1)
        process(t0 + 1, 1)
        return carry

    lax.fori_loop(0, (FULL - 1) // 2, pair, 0)
    wait(0, sa0, sb0)
    process(FULL - 1, 0)

    toff = FULL * CHUNK
    pltpu.async_copy(a_h.at[ibuf.at[pl.ds(toff, TAIL)]],
                     buf_a.at[pl.ds(0, TAIL)], sa0)
    pltpu.async_copy(b_h.at[cbuf.at[pl.ds(toff, TAIL)]],
                     buf_b.at[pl.ds(0, TAIL)], sb0)
    pltpu.make_async_copy(a_h.at[pl.ds(0, TAIL)], buf_a.at[pl.ds(0, TAIL)],
                          sa0).wait()
    pltpu.make_async_copy(b_h.at[pl.ds(0, TAIL)], buf_b.at[pl.ds(0, TAIL)],
                          sb0).wait()

    def add_row_t(r, inner):
        for f in range(HID // LANES):
            sl = pl.ds(f * LANES, LANES)
            buf_a[r, sl] = buf_a[r, sl] + buf_b[r, sl]
        return inner

    lax.fori_loop(0, TAIL, add_row_t, 0)
    pltpu.sync_copy(buf_a.at[pl.ds(0, TAIL)],
                    e1_h.at[pl.ds(base + toff, TAIL)])


_sc_edge = pl.kernel(
    _sc_edge_body,
    out_type=jax.ShapeDtypeStruct((N_EDGES, HID), jnp.float32),
    mesh=_mesh,
    scratch_types=[
        pltpu.VMEM((EPT,), jnp.int32),
        pltpu.VMEM((EPT,), jnp.int32),
        pltpu.VMEM((2 * CHUNK, HID), jnp.float32),
        pltpu.VMEM((2 * CHUNK, HID), jnp.float32),
        pltpu.SemaphoreType.DMA,
        pltpu.SemaphoreType.DMA,
        pltpu.SemaphoreType.DMA,
        pltpu.SemaphoreType.DMA,
    ],
)


def _sc_deg_body(ew_h, col_h, out_h, col2, tailc, ewb, zdbuf, obuf, sem,
                 shared):
    c = lax.axis_index("c")
    s = lax.axis_index("s")
    wid = s * NC + c
    base = wid * EPT
    zero16 = jnp.zeros((LANES,), jnp.float32)
    w = NPAD // NS

    def zbody(i, carry):
        zdbuf[pl.ds(i * LANES, LANES)] = zero16
        return carry

    lax.fori_loop(0, w // LANES, zbody, 0)
    pltpu.sync_copy(zdbuf, shared.at[pl.ds(s * w, w)])

    def ldfire(t, carry):
        pltpu.async_copy(col_h.at[pl.ds(base + t * CHUNK, CHUNK)],
                         col2.at[t], sem)
        return carry

    lax.fori_loop(0, FULL, ldfire, 0)
    pltpu.async_copy(col_h.at[pl.ds(base + FULL * CHUNK, TAIL)],
                     tailc.at[0], sem)
    pltpu.async_copy(ew_h.at[pl.ds(base, EPT)], ewb, sem)

    def lddrain(t, carry):
        pltpu.make_async_copy(col_h.at[pl.ds(0, CHUNK)], col2.at[0],
                              sem).wait()
        return carry

    lax.fori_loop(0, FULL, lddrain, 0)
    pltpu.make_async_copy(col_h.at[pl.ds(0, TAIL)], tailc.at[0], sem).wait()
    pltpu.make_async_copy(ew_h.at[pl.ds(0, EPT)], ewb, sem).wait()
    plsc.subcore_barrier()

    def fire(t, carry):
        pltpu.async_copy(ewb.at[pl.ds(t * CHUNK, CHUNK)],
                         shared.at[col2.at[t]], sem, add=True)
        return carry

    lax.fori_loop(0, FULL, fire, 0)
    pltpu.async_copy(ewb.at[pl.ds(FULL * CHUNK, TAIL)],
                     shared.at[tailc.at[0]], sem, add=True)

    def drain(t, carry):
        pltpu.make_async_copy(col_h.at[pl.ds(0, CHUNK)], col2.at[0],
                              sem).wait()
        return carry

    lax.fori_loop(0, FULL, drain, 0)
    pltpu.make_async_copy(col_h.at[pl.ds(0, TAIL)], tailc.at[0], sem).wait()
    plsc.subcore_barrier()

    @pl.when(s < 10)
    def _():
        pltpu.sync_copy(shared.at[pl.ds(s * 1000, 1000)], obuf)
        pltpu.sync_copy(obuf, out_h.at[pl.ds(c * N_NODES + s * 1000, 1000)])


_sc_deg = pl.kernel(
    _sc_deg_body,
    out_type=jax.ShapeDtypeStruct((NC * N_NODES,), jnp.float32),
    mesh=_mesh,
    scratch_types=[
        pltpu.VMEM((FULL, CHUNK), jnp.int32),
        pltpu.VMEM((1, TAIL), jnp.int32),
        pltpu.VMEM((EPT,), jnp.float32),
        pltpu.VMEM((NPAD // NS,), jnp.float32),
        pltpu.VMEM((1000,), jnp.float32),
        pltpu.SemaphoreType.DMA,
        pltpu.VMEM_SHARED((NPAD,), jnp.float32),
    ],
)


def _sc_msg_body(y_h, row_h, col_h, ew_h, out_h, ibuf, col2, tailc, ewb,
                 ybuf, sy0, sy1, shared):
    c = lax.axis_index("c")
    s = lax.axis_index("s")
    wid = s * NC + c
    base = wid * EPT
    zero16 = jnp.zeros((LANES,), jnp.float32)

    def zbody(r, carry):
        for f in range(HID // LANES):
            ybuf[r, pl.ds(f * LANES, LANES)] = zero16
        return carry

    lax.fori_loop(0, CHUNK, zbody, 0)

    @pl.when(s < 10)
    def _():
        for k in range(7):
            pltpu.sync_copy(ybuf.at[pl.ds(0, CHUNK)],
                            shared.at[pl.ds(s * 1000 + k * CHUNK, CHUNK)])
        pltpu.sync_copy(ybuf.at[pl.ds(0, 104)],
                        shared.at[pl.ds(s * 1000 + 896, 104)])

    pltpu.async_copy(row_h.at[pl.ds(base, EPT)], ibuf, sy0)

    def ldfire(t, carry):
        pltpu.async_copy(col_h.at[pl.ds(base + t * CHUNK, CHUNK)],
                         col2.at[t], sy0)
        return carry

    lax.fori_loop(0, FULL, ldfire, 0)
    pltpu.async_copy(col_h.at[pl.ds(base + FULL * CHUNK, TAIL)],
                     tailc.at[0], sy0)
    pltpu.async_copy(ew_h.at[pl.ds(base, EPT)], ewb.at[pl.ds(0, EPT)], sy0)
    pltpu.make_async_copy(row_h.at[pl.ds(0, EPT)], ibuf, sy0).wait()

    def lddrain(t, carry):
        pltpu.make_async_copy(col_h.at[pl.ds(0, CHUNK)], col2.at[0],
                              sy0).wait()
        return carry

    lax.fori_loop(0, FULL, lddrain, 0)
    pltpu.make_async_copy(col_h.at[pl.ds(0, TAIL)], tailc.at[0], sy0).wait()
    pltpu.make_async_copy(ew_h.at[pl.ds(0, EPT)], ewb.at[pl.ds(0, EPT)],
                          sy0).wait()
    plsc.subcore_barrier()

    def start(t, slot, sy):
        pltpu.async_copy(y_h.at[ibuf.at[pl.ds(t * CHUNK, CHUNK)]],
                         ybuf.at[pl.ds(slot * CHUNK, CHUNK)], sy)

    def wait(slot, sy):
        pltpu.make_async_copy(y_h.at[pl.ds(0, CHUNK)],
                              ybuf.at[pl.ds(slot * CHUNK, CHUNK)],
                              sy).wait()

    def process(t, slot):
        def scale_grp(g, inner):
            wgrp = ewb[pl.ds(t * CHUNK + g * LANES, LANES)]
            for l in range(LANES):
                w = wgrp[l]
                rr = slot * CHUNK + g * LANES + l
                for f in range(HID // LANES):
                    sl = pl.ds(f * LANES, LANES)
                    ybuf[rr, sl] = ybuf[rr, sl] * w
            return inner

        lax.fori_loop(0, CHUNK // LANES, scale_grp, 0)
        pltpu.sync_copy(ybuf.at[pl.ds(slot * CHUNK, CHUNK)],
                        shared.at[col2.at[t]], add=True)

    start(0, 0, sy0)

    def pair(u, carry):
        t0 = 2 * u
        start(t0 + 1, 1, sy1)
        wait(0, sy0)
        process(t0, 0)
        start(t0 + 2, 0, sy0)
        wait(1, sy1)
        process(t0 + 1, 1)
        return carry

    lax.fori_loop(0, (FULL - 1) // 2, pair, 0)
    wait(0, sy0)
    process(FULL - 1, 0)

    toff = FULL * CHUNK
    pltpu.async_copy(y_h.at[ibuf.at[pl.ds(toff, TAIL)]],
                     ybuf.at[pl.ds(0, TAIL)], sy0)
    pltpu.make_async_copy(y_h.at[pl.ds(0, TAIL)], ybuf.at[pl.ds(0, TAIL)],
                          sy0).wait()
    wgrp = ewb[pl.ds(toff, LANES)]
    for l in range(TAIL):
        w = wgrp[l]
        for f in range(HID // LANES):
            sl = pl.ds(f * LANES, LANES)
            ybuf[l, sl] = ybuf[l, sl] * w
    pltpu.sync_copy(ybuf.at[pl.ds(0, TAIL)], shared.at[tailc.at[0]],
                    add=True)
    plsc.subcore_barrier()

    @pl.when(s < 10)
    def _():
        for k in range(7):
            off = s * 1000 + k * CHUNK
            pltpu.sync_copy(shared.at[pl.ds(off, CHUNK)],
                            ybuf.at[pl.ds(0, CHUNK)])
            pltpu.sync_copy(ybuf.at[pl.ds(0, CHUNK)],
                            out_h.at[c, pl.ds(off, CHUNK)])
        off = s * 1000 + 896
        pltpu.sync_copy(shared.at[pl.ds(off, 104)], ybuf.at[pl.ds(0, 104)])
        pltpu.sync_copy(ybuf.at[pl.ds(0, 104)],
                        out_h.at[c, pl.ds(off, 104)])


_sc_msg = pl.kernel(
    _sc_msg_body,
    out_type=jax.ShapeDtypeStruct((NC, N_NODES, HID), jnp.float32),
    mesh=_mesh,
    scratch_types=[
        pltpu.VMEM((EPT,), jnp.int32),
        pltpu.VMEM((FULL, CHUNK), jnp.int32),
        pltpu.VMEM((1, TAIL), jnp.int32),
        pltpu.VMEM((EPT + LANES,), jnp.float32),
        pltpu.VMEM((2 * CHUNK, HID), jnp.float32),
        pltpu.SemaphoreType.DMA,
        pltpu.SemaphoreType.DMA,
        pltpu.VMEM_SHARED((N_NODES, HID), jnp.float32),
    ],
)


_RB = 2000


def _dot(a, b):
    return jnp.dot(a, b, preferred_element_type=jnp.float32,
                   precision=lax.Precision.HIGHEST)


def _tc1_body(x_ref, w1_ref, b1_ref, w2_ref, b2_ref, wa_ref, ba_ref, wb_ref,
              gw1_ref, h_ref, a_ref, b_ref, xw1_ref):
    x = x_ref[...]
    hm = jnp.maximum(_dot(x, w1_ref[...]) + b1_ref[...], 0.0)
    h = jnp.maximum(_dot(hm, w2_ref[...]) + b2_ref[...], 0.0)
    h_ref[...] = h
    a_ref[...] = _dot(h, wa_ref[...]) + ba_ref[...]
    b_ref[...] = _dot(h, wb_ref[...])
    xw1_ref[...] = _dot(h, gw1_ref[...])


def _tc1(x, w1f, b1f, w2, b2, wa, ba, wb, gw1):
    nb = N_NODES // _RB
    full = lambda shape: pl.BlockSpec(shape, lambda i: (0, 0))
    row_blk = lambda w: pl.BlockSpec((_RB, w), lambda i: (i, 0))
    return pl.pallas_call(
        _tc1_body,
        grid=(nb,),
        in_specs=[
            row_blk(HID), full((HID, 64)), full((1, 64)), full((64, HID)),
            full((1, HID)), full((HID, HID)), full((1, HID)),
            full((HID, HID)), full((HID, HID)),
        ],
        out_specs=[row_blk(HID), row_blk(HID), row_blk(HID), row_blk(HID)],
        out_shape=[jax.ShapeDtypeStruct((N_NODES, HID), jnp.float32)] * 4,
    )(x, w1f, b1f, w2, b2, wa, ba, wb, gw1)


_BE = 1280


def _tc2_body(e1_ref, ew2_ref, eb2_ref, ew3_ref, eb3_ref, out_ref):
    e1 = jnp.maximum(e1_ref[...], 0.0)
    e2 = jnp.maximum(
        jnp.dot(e1, ew2_ref[...], preferred_element_type=jnp.float32)
        + eb2_ref[...], 0.0)
    v = jnp.sum(e2 * ew3_ref[...], axis=1, keepdims=True) + eb3_ref[0, 0]
    out_ref[...] = jax.nn.sigmoid(v)


def _tc2(e1, ew2, eb2, ew3t, eb3):
    nb = N_EDGES // _BE
    full = lambda shape: pl.BlockSpec(shape, lambda i: (0, 0))
    return pl.pallas_call(
        _tc2_body,
        grid=(nb,),
        in_specs=[
            pl.BlockSpec((_BE, HID), lambda i: (i, 0)),
            full((HID, 64)), full((1, 64)), full((1, 64)), full((1, 1)),
        ],
        out_specs=pl.BlockSpec((_BE, 1), lambda i: (i, 0)),
        out_shape=jax.ShapeDtypeStruct((N_EDGES, 1), jnp.float32),
    )(e1, ew2, eb2, ew3t, eb3).reshape(N_EDGES)


def _tc3_body(p0_ref, p1_ref, xw1_ref, h_ref, gb1_ref, dinv_ref, y1_ref,
              aux1_ref):
    deg = 1.0 + p0_ref[...] + p1_ref[...]
    dinv = jnp.where(deg > 0, lax.rsqrt(jnp.maximum(deg, 1e-12)), 0.0)
    dinv_ref[...] = dinv
    xw = xw1_ref[...]
    y1_ref[...] = dinv * xw
    aux1_ref[...] = h_ref[...] + gb1_ref[...] + dinv * dinv * xw


def _tc3(p0, p1, xw1, h, gb1):
    nb = N_NODES // _RB
    col_blk = pl.BlockSpec((_RB, 1), lambda i: (i, 0))
    row_blk = pl.BlockSpec((_RB, HID), lambda i: (i, 0))
    full = lambda shape: pl.BlockSpec(shape, lambda i: (0, 0))
    return pl.pallas_call(
        _tc3_body,
        grid=(nb,),
        in_specs=[col_blk, col_blk, row_blk, row_blk, full((1, HID))],
        out_specs=[col_blk, row_blk, row_blk],
        out_shape=[
            jax.ShapeDtypeStruct((N_NODES, 1), jnp.float32),
            jax.ShapeDtypeStruct((N_NODES, HID), jnp.float32),
            jax.ShapeDtypeStruct((N_NODES, HID), jnp.float32),
        ],
    )(p0, p1, xw1, h, gb1)


def _tc4_body(aux1_ref, q0_ref, q1_ref, dinv_ref, gw2_ref, gb2_ref, y2_ref,
              aux2_ref):
    dinv = dinv_ref[...]
    h1 = jnp.maximum(aux1_ref[...] + dinv * (q0_ref[...] + q1_ref[...]), 0.0)
    xw2 = _dot(h1, gw2_ref[...])
    y2_ref[...] = dinv * xw2
    aux2_ref[...] = h1 + gb2_ref[...] + dinv * dinv * xw2


def _tc4(aux1, q0, q1, dinv, gw2, gb2):
    nb = N_NODES // _RB
    col_blk = pl.BlockSpec((_RB, 1), lambda i: (i, 0))
    row_blk = pl.BlockSpec((_RB, HID), lambda i: (i, 0))
    full = lambda shape: pl.BlockSpec(shape, lambda i: (0, 0))
    return pl.pallas_call(
        _tc4_body,
        grid=(nb,),
        in_specs=[row_blk, row_blk, row_blk, col_blk, full((HID, HID)),
                  full((1, HID))],
        out_specs=[row_blk, row_blk],
        out_shape=[jax.ShapeDtypeStruct((N_NODES, HID), jnp.float32)] * 2,
    )(aux1, q0, q1, dinv, gw2, gb2)


def _tc5_body(aux2_ref, r0_ref, r1_ref, dinv_ref, pw1_ref, pb1_ref, pw2_ref,
              pb2_ref, pw3_ref, pb3_ref, out_ref, sacc, macc):
    i = pl.program_id(0)
    dinv = dinv_ref[...]
    h2 = jnp.maximum(aux2_ref[...] + dinv * (r0_ref[...] + r1_ref[...]), 0.0)

    @pl.when(i == 0)
    def _():
        sacc[...] = jnp.zeros_like(sacc)
        macc[...] = jnp.full_like(macc, -jnp.inf)

    sacc[...] = sacc[...] + jnp.sum(h2, axis=0, keepdims=True)
    macc[...] = jnp.maximum(macc[...], jnp.max(h2, axis=0, keepdims=True))

    @pl.when(i == pl.num_programs(0) - 1)
    def _():
        g = (sacc[...] / N_NODES + macc[...]) * 0.5
        o1 = jnp.maximum(_dot(g, pw1_ref[...]) + pb1_ref[...], 0.0)
        o2 = jnp.maximum(_dot(o1, pw2_ref[...]) + pb2_ref[...], 0.0)
        out_ref[...] = _dot(o2, pw3_ref[...]) + pb3_ref[...]


def _tc5(aux2, r0, r1, dinv, pw1, pb1, pw2, pb2, pw3, pb3):
    nb = N_NODES // _RB
    col_blk = pl.BlockSpec((_RB, 1), lambda i: (i, 0))
    row_blk = pl.BlockSpec((_RB, HID), lambda i: (i, 0))
    full = lambda shape: pl.BlockSpec(shape, lambda i: (0, 0))
    return pl.pallas_call(
        _tc5_body,
        grid=(nb,),
        in_specs=[row_blk, row_blk, row_blk, col_blk,
                  full((HID, 64)), full((1, 64)), full((64, 32)),
                  full((1, 32)), full((32, 1)), full((1, 1))],
        out_specs=pl.BlockSpec((1, 1), lambda i: (0, 0)),
        out_shape=jax.ShapeDtypeStruct((1, 1), jnp.float32),
        scratch_shapes=[
            pltpu.VMEM((1, HID), jnp.float32),
            pltpu.VMEM((1, HID), jnp.float32),
        ],
    )(aux2, r0, r1, dinv, pw1, pb1, pw2, pb2, pw3, pb3)


def kernel(x, edge_index, batch, W1, b1, g1, be1, W2, b2, Ew1, Eb1, Eg, Ebe,
           Ew2, Eb2, Ew3, Eb3, GW1, Gb1, GW2, Gb2, Pw1, Pb1, Pw2, Pb2, Pw3,
           Pb3):
    s = 1.0 / jnp.sqrt(jnp.float32(1.0 + 1e-5))
    w1f = W1 * (g1 * s)[None, :]
    b1f = (b1 * g1 * s + be1)[None, :]
    ewa = Ew1[:HID] * (Eg * s)[None, :]
    ewb = Ew1[HID:] * (Eg * s)[None, :]
    ebf = (Eb1 * Eg * s + Ebe)[None, :]
    row = edge_index[0]
    col = edge_index[1]

    h, a_t, b_t, xw1 = _tc1(x, w1f, b1f, W2, b2[None], ewa, ebf, ewb, GW1)
    e1 = _sc_edge(a_t, b_t, row, col)
    ew = _tc2(e1, Ew2, Eb2[None], Ew3.T, Eb3[None])
    degp = _sc_deg(ew, col)
    p0 = degp[:N_NODES][:, None]
    p1 = degp[N_NODES:][:, None]
    dinv, y1, aux1 = _tc3(p0, p1, xw1, h, Gb1[None])
    q = _sc_msg(y1, row, col, ew)
    y2, aux2 = _tc4(aux1, q[0], q[1], dinv, GW2, Gb2[None])
    r = _sc_msg(y2, row, col, ew)
    return _tc5(aux2, r[0], r[1], dinv, Pw1, Pb1[None], Pw2, Pb2[None], Pw3,
                Pb3[None])
